# baseline (speedup 1.0000x reference)
import functools

import jax
import jax.numpy as jnp
from jax import lax
from jax.experimental import pallas as pl
from jax.experimental.pallas import tpu as pltpu
from jax.experimental.pallas import tpu_sc as plsc

_NC = 2
_NS = 16
_NW = _NC * _NS
_L = 16

_B = 16384
_D = 64
_V = 1000001
_SUP = 256
_NBLK = 3906
_TAIL_OFF = 999936
_RPT = 123
_NPAIR = (_RPT + 1) // 2
_NVEC = _B // _L
_ORING = 32


@functools.partial(
    pl.kernel,
    out_type=jax.ShapeDtypeStruct((_B, _D), jnp.float32),
    mesh=plsc.VectorSubcoreMesh(core_axis_name="c", subcore_axis_name="s"),
    scratch_types=[
        pltpu.VMEM((_B,), jnp.int32),
        pltpu.VMEM((_B + _L,), jnp.int32),
        pltpu.VMEM((_B + _L,), jnp.int32),
        pltpu.VMEM((2, _D, _SUP), jnp.float32),
        pltpu.VMEM((_D, 128), jnp.float32),
        pltpu.VMEM((_ORING, _D), jnp.float32),
        pltpu.VMEM((_L,), jnp.int32),
        pltpu.SemaphoreType.DMA,
        pltpu.SemaphoreType.DMA,
        pltpu.SemaphoreType.DMA,
    ],
    compiler_params=pltpu.CompilerParams(needs_layout_passes=False),
)
def _embed_lookup(labels_hbm, table_t_hbm, out_hbm, lab_v, list_v, pos_v,
                  blk_v, tail_v, outb_v, ocnt_v, sem0, sem1, sem_o):
    wid = lax.axis_index("s") * _NC + lax.axis_index("c")
    lanes = lax.broadcasted_iota(jnp.int32, (_L,), 0)
    zeros = jnp.zeros((_L,), jnp.int32)
    lane0 = lanes == 0
    rows = [lanes + q * _L for q in range(_D // _L)]

    ocnt_v[pl.ds(0, _L)] = zeros
    pltpu.sync_copy(labels_hbm, lab_v)

    wid_v = jnp.full((_L,), wid, jnp.int32)

    def dist_body(v, off):
        vec = lab_v[pl.ds(v * _L, _L)]
        mine = (vec // _SUP) // _RPT == wid_v
        plsc.store_compressed(list_v.at[pl.ds(off, _L)], vec, mask=mine)
        plsc.store_compressed(pos_v.at[pl.ds(off, _L)], lanes + v * _L, mask=mine)
        cnt = plsc.all_reduce_population_count(mine)
        return off + cnt[0]

    n_mine = lax.fori_loop(0, _NVEC, dist_body, 0)
    nvec_mine = (n_mine + _L - 1) // _L
    valid_limit = jnp.full((_L,), n_mine, jnp.int32)

    def drain_one():
        pltpu.make_async_copy(
            out_hbm.at[pl.ds(0, 1), :], outb_v.at[pl.ds(0, 1), :], sem_o
        ).wait()

    def process_hits(b, block):
        b_v = jnp.full((_L,), b, jnp.int32)

        def scan_list(v, carry):
            lv = list_v[pl.ds(v * _L, _L)]
            valid = (lanes + v * _L) < valid_limit
            hm = (lv // _SUP == b_v) & valid
            nh = plsc.all_reduce_population_count(hm)[0]

            def hit_body(j, hm):
                l_v = plsc.all_reduce_ffs(hm)
                l_v = jnp.broadcast_to(l_v, (_L,)).astype(jnp.int32)
                label_v = plsc.load_gather(list_v, [l_v + v * _L])
                p_v = plsc.load_gather(pos_v, [l_v + v * _L])
                lane_v = label_v % _SUP
                no = plsc.load_gather(ocnt_v, [zeros])[0]

                @pl.when(no >= _ORING)
                def _():
                    drain_one()

                slot = no % _ORING
                for q in range(_D // _L):
                    vals = plsc.load_gather(block, [rows[q], lane_v])
                    outb_v[slot, pl.ds(q * _L, _L)] = vals
                pltpu.async_copy(
                    outb_v.at[pl.ds(slot, 1), :],
                    out_hbm.at[pl.ds(p_v[0], 1), :],
                    sem_o,
                )
                plsc.store_scatter(ocnt_v, [zeros], jnp.full(
                    (_L,), no + 1, jnp.int32), mask=lane0)
                return hm & (lanes != l_v)

            lax.fori_loop(0, nh, hit_body, hm)
            return carry

        lax.fori_loop(0, nvec_mine, scan_list, 0)

    start = wid * _RPT
    nb = jnp.minimum(_RPT, _NBLK - start)

    def fetch_blk(bb, buf, sem):
        off = pl.multiple_of((start + bb) * _SUP, 128)
        pltpu.async_copy(table_t_hbm.at[:, pl.ds(off, _SUP)],
                         blk_v.at[buf], sem)

    fetch_blk(0, 0, sem0)

    def pair_body(p, carry):
        for r2 in range(2):
            bb = p * 2 + r2
            sem = sem0 if r2 == 0 else sem1
            other = sem1 if r2 == 0 else sem0

            @pl.when(bb < nb)
            def _():
                @pl.when(bb + 1 < nb)
                def _():
                    fetch_blk(bb + 1, 1 - r2, other)

                pltpu.make_async_copy(
                    table_t_hbm.at[:, pl.ds(0, _SUP)], blk_v.at[r2], sem
                ).wait()
                process_hits(start + bb, blk_v.at[r2])
        return carry

    lax.fori_loop(0, _NPAIR, pair_body, 0)

    @pl.when(wid == _NW - 1)
    def _():
        off = pl.multiple_of(_TAIL_OFF, 128)
        pltpu.async_copy(table_t_hbm.at[:, pl.ds(off, 128)], tail_v, sem0)
        pltpu.make_async_copy(
            table_t_hbm.at[:, pl.ds(0, 128)], tail_v, sem0
        ).wait()
        process_hits(_NBLK, tail_v)

    n_out = plsc.load_gather(ocnt_v, [zeros])[0]

    def drain_body(j, carry):
        drain_one()
        return carry

    lax.fori_loop(0, jnp.minimum(n_out, _ORING), drain_body, 0)


def kernel(labels, table):
    return _embed_lookup(labels.astype(jnp.int32), table.T)


# speedup vs baseline: 1.0403x; 1.0403x over previous
import functools

import jax
import jax.numpy as jnp
from jax import lax
from jax.experimental import pallas as pl
from jax.experimental.pallas import tpu as pltpu
from jax.experimental.pallas import tpu_sc as plsc

_NC = 2
_NS = 16
_NW = _NC * _NS
_L = 16

_B = 16384
_D = 64
_V = 1000001
_SUP = 256
_NBLK = 3906
_TAIL_OFF = 999936
_RPT = 123
_NPAIR = (_RPT + 1) // 2
_NVEC = _B // _L
_ORING = 32


@functools.partial(
    pl.kernel,
    out_type=jax.ShapeDtypeStruct((_B, _D), jnp.float32),
    mesh=plsc.VectorSubcoreMesh(core_axis_name="c", subcore_axis_name="s"),
    scratch_types=[
        pltpu.VMEM((_B,), jnp.int32),
        pltpu.VMEM((_B + _L,), jnp.int32),
        pltpu.VMEM((_B + _L,), jnp.int32),
        pltpu.VMEM((4, _D, _SUP), jnp.float32),
        pltpu.VMEM((_D, 128), jnp.float32),
        pltpu.VMEM((_ORING, _D), jnp.float32),
        pltpu.VMEM((_L,), jnp.int32),
        pltpu.SemaphoreType.DMA,
        pltpu.SemaphoreType.DMA,
        pltpu.SemaphoreType.DMA,
        pltpu.SemaphoreType.DMA,
        pltpu.SemaphoreType.DMA,
    ],
    compiler_params=pltpu.CompilerParams(needs_layout_passes=False),
)
def _embed_lookup(labels_hbm, table_t_hbm, out_hbm, lab_v, list_v, pos_v,
                  blk_v, tail_v, outb_v, ocnt_v, sem0, sem1, sem2, sem3,
                  sem_o):
    bsems = (sem0, sem1, sem2, sem3)
    wid = lax.axis_index("s") * _NC + lax.axis_index("c")
    lanes = lax.broadcasted_iota(jnp.int32, (_L,), 0)
    zeros = jnp.zeros((_L,), jnp.int32)
    lane0 = lanes == 0
    rows = [lanes + q * _L for q in range(_D // _L)]

    ocnt_v[pl.ds(0, _L)] = zeros
    pltpu.sync_copy(labels_hbm, lab_v)

    wid_v = jnp.full((_L,), wid, jnp.int32)

    def dist_body(v, off):
        vec = lab_v[pl.ds(v * _L, _L)]
        mine = (vec // _SUP) // _RPT == wid_v
        plsc.store_compressed(list_v.at[pl.ds(off, _L)], vec, mask=mine)
        plsc.store_compressed(pos_v.at[pl.ds(off, _L)], lanes + v * _L, mask=mine)
        cnt = plsc.all_reduce_population_count(mine)
        return off + cnt[0]

    n_mine = lax.fori_loop(0, _NVEC, dist_body, 0)
    nvec_mine = (n_mine + _L - 1) // _L
    valid_limit = jnp.full((_L,), n_mine, jnp.int32)

    def drain_one():
        pltpu.make_async_copy(
            out_hbm.at[pl.ds(0, 1), :], outb_v.at[pl.ds(0, 1), :], sem_o
        ).wait()

    def process_hits(b, block):
        b_v = jnp.full((_L,), b, jnp.int32)

        def scan_list(v, carry):
            lv = list_v[pl.ds(v * _L, _L)]
            valid = (lanes + v * _L) < valid_limit
            hm = (lv // _SUP == b_v) & valid
            nh = plsc.all_reduce_population_count(hm)[0]

            def hit_body(j, hm):
                l_v = plsc.all_reduce_ffs(hm)
                l_v = jnp.broadcast_to(l_v, (_L,)).astype(jnp.int32)
                label_v = plsc.load_gather(list_v, [l_v + v * _L])
                p_v = plsc.load_gather(pos_v, [l_v + v * _L])
                lane_v = label_v % _SUP
                no = plsc.load_gather(ocnt_v, [zeros])[0]

                @pl.when(no >= _ORING)
                def _():
                    drain_one()

                slot = no % _ORING
                for q in range(_D // _L):
                    vals = plsc.load_gather(block, [rows[q], lane_v])
                    outb_v[slot, pl.ds(q * _L, _L)] = vals
                pltpu.async_copy(
                    outb_v.at[pl.ds(slot, 1), :],
                    out_hbm.at[pl.ds(p_v[0], 1), :],
                    sem_o,
                )
                plsc.store_scatter(ocnt_v, [zeros], jnp.full(
                    (_L,), no + 1, jnp.int32), mask=lane0)
                return hm & (lanes != l_v)

            lax.fori_loop(0, nh, hit_body, hm)
            return carry

        lax.fori_loop(0, nvec_mine, scan_list, 0)

    start = wid * _RPT
    nb = jnp.minimum(_RPT, _NBLK - start)

    def fetch_blk(bb, buf, sem):
        off = pl.multiple_of((start + bb) * _SUP, 128)
        pltpu.async_copy(table_t_hbm.at[:, pl.ds(off, _SUP)],
                         blk_v.at[buf], sem)

    for k in range(3):
        fetch_blk(k, k, bsems[k])

    def quad_body(p, carry):
        for r4 in range(4):
            bb = p * 4 + r4

            @pl.when(bb < nb)
            def _():
                @pl.when(bb + 3 < nb)
                def _():
                    fetch_blk(bb + 3, (r4 + 3) % 4, bsems[(r4 + 3) % 4])

                pltpu.make_async_copy(
                    table_t_hbm.at[:, pl.ds(0, _SUP)], blk_v.at[r4],
                    bsems[r4]
                ).wait()
                process_hits(start + bb, blk_v.at[r4])
        return carry

    lax.fori_loop(0, (_RPT + 3) // 4, quad_body, 0)

    @pl.when(wid == _NW - 1)
    def _():
        off = pl.multiple_of(_TAIL_OFF, 128)
        pltpu.async_copy(table_t_hbm.at[:, pl.ds(off, 128)], tail_v, sem0)
        pltpu.make_async_copy(
            table_t_hbm.at[:, pl.ds(0, 128)], tail_v, sem0
        ).wait()
        process_hits(_NBLK, tail_v)

    n_out = plsc.load_gather(ocnt_v, [zeros])[0]

    def drain_body(j, carry):
        drain_one()
        return carry

    lax.fori_loop(0, jnp.minimum(n_out, _ORING), drain_body, 0)


def kernel(labels, table):
    return _embed_lookup(labels.astype(jnp.int32), table.T)


# speedup vs baseline: 1.3346x; 1.2830x over previous
import functools

import jax
import jax.numpy as jnp
from jax import lax
from jax.experimental import pallas as pl
from jax.experimental.pallas import tpu as pltpu
from jax.experimental.pallas import tpu_sc as plsc

_NC = 2
_NS = 16
_NW = _NC * _NS
_L = 16

_B = 16384
_D = 64
_SUP = 256
_NBLK = 3906
_TAIL_OFF = 999936
_RPT = 123
_NVEC = _B // _L
_K = 8
_ORING = 32
_PSH = 14


@functools.partial(
    pl.kernel,
    out_type=jax.ShapeDtypeStruct((_B, _D), jnp.float32),
    mesh=plsc.VectorSubcoreMesh(core_axis_name="c", subcore_axis_name="s"),
    scratch_types=[
        pltpu.VMEM((_B + _L,), jnp.int32),
        pltpu.VMEM((_B + _L,), jnp.int32),
        pltpu.VMEM((4, _D, _SUP), jnp.float32),
        pltpu.VMEM((_D, 128), jnp.float32),
        pltpu.VMEM((_ORING, _D), jnp.float32),
        pltpu.VMEM((_L,), jnp.int32),
        pltpu.VMEM((_L,), jnp.int32),
        pltpu.SemaphoreType.DMA,
        pltpu.SemaphoreType.DMA,
        pltpu.SemaphoreType.DMA,
        pltpu.SemaphoreType.DMA,
        pltpu.SemaphoreType.DMA,
    ],
    compiler_params=pltpu.CompilerParams(needs_layout_passes=False),
)
def _embed_lookup(labels_hbm, table_t_hbm, out_hbm, lab_v, pk1_v,
                  blk_v, tail_v, outb_v, ocnt_v, starts_v,
                  sem0, sem1, sem2, sem3, sem_o):
    bsems = (sem0, sem1, sem2, sem3)
    wid = lax.axis_index("s") * _NC + lax.axis_index("c")
    lanes = lax.broadcasted_iota(jnp.int32, (_L,), 0)
    zeros = jnp.zeros((_L,), jnp.int32)
    lane0 = lanes == 0
    rows = [lanes + q * _L for q in range(_D // _L)]

    ocnt_v[pl.ds(0, _L)] = zeros
    starts_v[pl.ds(0, _L)] = zeros
    pltpu.sync_copy(labels_hbm, lab_v.at[pl.ds(0, _B)])

    start = wid * _RPT
    base_row = start * _SUP

    wid_v = jnp.full((_L,), wid, jnp.int32)

    def dist_body(v, off):
        vec = lab_v[pl.ds(v * _L, _L)]
        mine = (vec // _SUP) // _RPT == wid_v
        packed = ((vec - base_row) << _PSH) | (lanes + v * _L)
        plsc.store_compressed(pk1_v.at[pl.ds(off, _L)], packed, mask=mine)
        return off + plsc.all_reduce_population_count(mine)[0]

    n_mine = lax.fori_loop(0, _NVEC, dist_body, 0)
    nvec_mine = (n_mine + _L - 1) // _L
    n_mine_v = jnp.full((_L,), n_mine, jnp.int32)

    def part_one(s):
        s_v = jnp.full((_L,), s, jnp.int32)

        def body(v, off):
            pv = pk1_v[pl.ds(v * _L, _L)]
            valid = (lanes + v * _L) < n_mine_v
            m = (((pv >> _PSH) // (_SUP * 16)) == s_v) & valid
            plsc.store_compressed(lab_v.at[pl.ds(off, _L)], pv, mask=m)
            return off + plsc.all_reduce_population_count(m)[0]

        return body

    off2 = 0
    for s in range(_K):
        off2 = lax.fori_loop(0, nvec_mine, part_one(s), off2)
        plsc.store_scatter(
            starts_v, [jnp.full((_L,), s + 1, jnp.int32)],
            jnp.full((_L,), off2, jnp.int32), mask=lane0)

    def drain_one():
        pltpu.make_async_copy(
            out_hbm.at[pl.ds(0, 1), :], outb_v.at[pl.ds(0, 1), :], sem_o
        ).wait()

    def process_hits(bb, block):
        s = bb // 16
        st = plsc.load_gather(starts_v, [jnp.full((_L,), s, jnp.int32)])
        en = plsc.load_gather(starts_v, [jnp.full((_L,), s + 1, jnp.int32)])
        bb_v = jnp.full((_L,), bb, jnp.int32)

        def scan_list(v, carry):
            pv = lab_v[pl.ds(v * _L, _L)]
            gidx = lanes + v * _L
            valid = (gidx >= st) & (gidx < en)
            hm = (((pv >> _PSH) // _SUP) == bb_v) & valid
            nh = plsc.all_reduce_population_count(hm)[0]

            def hit_body(j, hm):
                l_v = plsc.all_reduce_ffs(hm)
                l_v = jnp.broadcast_to(l_v, (_L,)).astype(jnp.int32)
                pk = plsc.load_gather(lab_v, [l_v + v * _L])
                lane_v = (pk >> _PSH) % _SUP
                pos = (pk & ((1 << _PSH) - 1))[0]
                no = plsc.load_gather(ocnt_v, [zeros])[0]

                @pl.when(no >= _ORING)
                def _():
                    drain_one()

                slot = no % _ORING
                for q in range(_D // _L):
                    vals = plsc.load_gather(block, [rows[q], lane_v])
                    outb_v[slot, pl.ds(q * _L, _L)] = vals
                pltpu.async_copy(
                    outb_v.at[pl.ds(slot, 1), :],
                    out_hbm.at[pl.ds(pos, 1), :],
                    sem_o,
                )
                plsc.store_scatter(ocnt_v, [zeros], jnp.full(
                    (_L,), no + 1, jnp.int32), mask=lane0)
                return hm & (lanes != l_v)

            lax.fori_loop(0, nh, hit_body, hm)
            return carry

        lax.fori_loop(st[0] // _L, (en[0] + _L - 1) // _L, scan_list, 0)

    nb = jnp.minimum(_RPT, _NBLK - start)

    def fetch_blk(bb, buf, sem):
        off = pl.multiple_of((start + bb) * _SUP, 128)
        pltpu.async_copy(table_t_hbm.at[:, pl.ds(off, _SUP)],
                         blk_v.at[buf], sem)

    for k in range(3):
        fetch_blk(k, k, bsems[k])

    def quad_body(p, carry):
        for r4 in range(4):
            bb = p * 4 + r4

            @pl.when(bb < nb)
            def _():
                @pl.when(bb + 3 < nb)
                def _():
                    fetch_blk(bb + 3, (r4 + 3) % 4, bsems[(r4 + 3) % 4])

                pltpu.make_async_copy(
                    table_t_hbm.at[:, pl.ds(0, _SUP)], blk_v.at[r4],
                    bsems[r4]
                ).wait()
                process_hits(bb, blk_v.at[r4])
        return carry

    lax.fori_loop(0, (_RPT + 3) // 4, quad_body, 0)

    @pl.when(wid == _NW - 1)
    def _():
        off = pl.multiple_of(_TAIL_OFF, 128)
        pltpu.async_copy(table_t_hbm.at[:, pl.ds(off, 128)], tail_v, sem0)
        pltpu.make_async_copy(
            table_t_hbm.at[:, pl.ds(0, 128)], tail_v, sem0
        ).wait()
        process_hits(_NBLK - start, tail_v)

    n_out = plsc.load_gather(ocnt_v, [zeros])[0]

    def drain_body(j, carry):
        drain_one()
        return carry

    lax.fori_loop(0, jnp.minimum(n_out, _ORING), drain_body, 0)


def kernel(labels, table):
    return _embed_lookup(labels.astype(jnp.int32), table.T)


# speedup vs baseline: 1.3821x; 1.0356x over previous
import functools

import jax
import jax.numpy as jnp
from jax import lax
from jax.experimental import pallas as pl
from jax.experimental.pallas import tpu as pltpu
from jax.experimental.pallas import tpu_sc as plsc

_NC = 2
_NS = 16
_NW = _NC * _NS
_L = 16

_B = 16384
_D = 64
_SUP = 256
_NBLK = 3906
_TAIL_OFF = 999936
_RPT = 123
_NVEC = _B // _L
_K = 8
_ORING = 32
_PSH = 14


@functools.partial(
    pl.kernel,
    out_type=jax.ShapeDtypeStruct((_B, _D), jnp.float32),
    mesh=plsc.VectorSubcoreMesh(core_axis_name="c", subcore_axis_name="s"),
    scratch_types=[
        pltpu.VMEM((_B + _L,), jnp.int32),
        pltpu.VMEM((_B + _L,), jnp.int32),
        pltpu.VMEM((5, _D, _SUP), jnp.float32),
        pltpu.VMEM((_D, 128), jnp.float32),
        pltpu.VMEM((_ORING, _D), jnp.float32),
        pltpu.VMEM((_L,), jnp.int32),
        pltpu.VMEM((_L,), jnp.int32),
        pltpu.SemaphoreType.DMA,
        pltpu.SemaphoreType.DMA,
        pltpu.SemaphoreType.DMA,
        pltpu.SemaphoreType.DMA,
        pltpu.SemaphoreType.DMA,
        pltpu.SemaphoreType.DMA,
    ],
    compiler_params=pltpu.CompilerParams(needs_layout_passes=False),
)
def _embed_lookup(labels_hbm, table_t_hbm, out_hbm, lab_v, pk1_v,
                  blk_v, tail_v, outb_v, ocnt_v, starts_v,
                  sem0, sem1, sem2, sem3, sem4, sem_o):
    bsems = (sem0, sem1, sem2, sem3, sem4)
    wid = lax.axis_index("s") * _NC + lax.axis_index("c")
    lanes = lax.broadcasted_iota(jnp.int32, (_L,), 0)
    zeros = jnp.zeros((_L,), jnp.int32)
    lane0 = lanes == 0
    rows = [lanes + q * _L for q in range(_D // _L)]

    ocnt_v[pl.ds(0, _L)] = zeros
    starts_v[pl.ds(0, _L)] = zeros
    pltpu.sync_copy(labels_hbm, lab_v.at[pl.ds(0, _B)])

    start = wid * _RPT
    base_row = start * _SUP

    wid_v = jnp.full((_L,), wid, jnp.int32)

    def dist_body(v, off):
        vec0 = lab_v[pl.ds(v * 2 * _L, _L)]
        vec1 = lab_v[pl.ds((v * 2 + 1) * _L, _L)]
        m0 = vec0 // (_SUP * _RPT) == wid_v
        m1 = vec1 // (_SUP * _RPT) == wid_v
        p0 = ((vec0 - base_row) << _PSH) | (lanes + v * 2 * _L)
        p1 = ((vec1 - base_row) << _PSH) | (lanes + (v * 2 + 1) * _L)
        c0 = plsc.all_reduce_population_count(m0)[0]
        c1 = plsc.all_reduce_population_count(m1)[0]
        plsc.store_compressed(pk1_v.at[pl.ds(off, _L)], p0, mask=m0)
        plsc.store_compressed(pk1_v.at[pl.ds(off + c0, _L)], p1, mask=m1)
        return off + c0 + c1

    n_mine = lax.fori_loop(0, _NVEC // 2, dist_body, 0)
    nvec_mine = (n_mine + _L - 1) // _L
    n_mine_v = jnp.full((_L,), n_mine, jnp.int32)

    def part_one(s):
        s_v = jnp.full((_L,), s, jnp.int32)

        def body(v, off):
            pv = pk1_v[pl.ds(v * _L, _L)]
            valid = (lanes + v * _L) < n_mine_v
            m = ((pv >> (_PSH + 12)) == s_v) & valid
            plsc.store_compressed(lab_v.at[pl.ds(off, _L)], pv, mask=m)
            return off + plsc.all_reduce_population_count(m)[0]

        return body

    off2 = 0
    for s in range(_K):
        off2 = lax.fori_loop(0, nvec_mine, part_one(s), off2)
        plsc.store_scatter(
            starts_v, [jnp.full((_L,), s + 1, jnp.int32)],
            jnp.full((_L,), off2, jnp.int32), mask=lane0)

    def drain_one():
        pltpu.make_async_copy(
            out_hbm.at[pl.ds(0, 1), :], outb_v.at[pl.ds(0, 1), :], sem_o
        ).wait()

    def process_hits(bb, block):
        s = bb // 16
        st = plsc.load_gather(starts_v, [jnp.full((_L,), s, jnp.int32)])
        en = plsc.load_gather(starts_v, [jnp.full((_L,), s + 1, jnp.int32)])
        bb_v = jnp.full((_L,), bb, jnp.int32)

        def scan_list(v, carry):
            pv = lab_v[pl.ds(v * _L, _L)]
            gidx = lanes + v * _L
            valid = (gidx >= st) & (gidx < en)
            hm = ((pv >> (_PSH + 8)) == bb_v) & valid
            nh = plsc.all_reduce_population_count(hm)[0]

            def hit_body(j, hm):
                l_v = plsc.all_reduce_ffs(hm)
                l_v = jnp.broadcast_to(l_v, (_L,)).astype(jnp.int32)
                pk = plsc.load_gather(lab_v, [l_v + v * _L])
                lane_v = (pk >> _PSH) & (_SUP - 1)
                pos = (pk & ((1 << _PSH) - 1))[0]
                no = plsc.load_gather(ocnt_v, [zeros])[0]

                @pl.when(no >= _ORING)
                def _():
                    drain_one()

                slot = no % _ORING
                for q in range(_D // _L):
                    vals = plsc.load_gather(block, [rows[q], lane_v])
                    outb_v[slot, pl.ds(q * _L, _L)] = vals
                pltpu.async_copy(
                    outb_v.at[pl.ds(slot, 1), :],
                    out_hbm.at[pl.ds(pos, 1), :],
                    sem_o,
                )
                plsc.store_scatter(ocnt_v, [zeros], jnp.full(
                    (_L,), no + 1, jnp.int32), mask=lane0)
                return hm & (lanes != l_v)

            lax.fori_loop(0, nh, hit_body, hm)
            return carry

        lax.fori_loop(st[0] // _L, (en[0] + _L - 1) // _L, scan_list, 0)

    nb = jnp.minimum(_RPT, _NBLK - start)

    def fetch_blk(bb, buf, sem):
        off = pl.multiple_of((start + bb) * _SUP, 128)
        pltpu.async_copy(table_t_hbm.at[:, pl.ds(off, _SUP)],
                         blk_v.at[buf], sem)

    for k in range(4):
        fetch_blk(k, k, bsems[k])

    def ring_body(p, carry):
        for r5 in range(5):
            bb = p * 5 + r5

            @pl.when(bb < nb)
            def _():
                @pl.when(bb + 4 < nb)
                def _():
                    fetch_blk(bb + 4, (r5 + 4) % 5, bsems[(r5 + 4) % 5])

                pltpu.make_async_copy(
                    table_t_hbm.at[:, pl.ds(0, _SUP)], blk_v.at[r5],
                    bsems[r5]
                ).wait()
                process_hits(bb, blk_v.at[r5])
        return carry

    lax.fori_loop(0, (_RPT + 4) // 5, ring_body, 0)

    @pl.when(wid == _NW - 1)
    def _():
        off = pl.multiple_of(_TAIL_OFF, 128)
        pltpu.async_copy(table_t_hbm.at[:, pl.ds(off, 128)], tail_v, sem0)
        pltpu.make_async_copy(
            table_t_hbm.at[:, pl.ds(0, 128)], tail_v, sem0
        ).wait()
        process_hits(_NBLK - start, tail_v)

    n_out = plsc.load_gather(ocnt_v, [zeros])[0]

    def drain_body(j, carry):
        drain_one()
        return carry

    lax.fori_loop(0, jnp.minimum(n_out, _ORING), drain_body, 0)


def kernel(labels, table):
    return _embed_lookup(labels.astype(jnp.int32), table.T)
